# Initial kernel scaffold; baseline (speedup 1.0000x reference)
import jax
import jax.numpy as jnp
from jax.experimental import pallas as pl


def kernel(x, eis, eas, idxs, ptrs, W1, b1, W2, b2, W3, b3, A1, a1, A2, a2, root, bc4, Wi, bi, Wh, bh, Wl, bl, Wd, bd):
    raise NotImplementedError("write your pallas kernel here")


# speedup vs baseline: 4.0487x; 4.0487x over previous
import jax
import jax.numpy as jnp
from jax.experimental import pallas as pl

N = 10000
E = 160000
H = 32
S = 5
C = 128
K = 79


def _gru_step(h, g, Wh, bh):
    gh = h @ Wh.T + bh
    ir, iz, i_n = jnp.split(g, 3, axis=1)
    hr, hz, hn = jnp.split(gh, 3, axis=1)
    r = jax.nn.sigmoid(ir + hr)
    zg = jax.nn.sigmoid(iz + hz)
    ng = jnp.tanh(i_n + r * hn)
    return (1.0 - zg) * ng + zg * h


def kernel(x, eis, eas, idxs, ptrs, W1, b1, W2, b2, W3, b3, A1, a1, A2, a2,
           root, bc4, Wi, bi, Wh, bh, Wl, bl, Wd, bd):
    ei = eis[0]
    ea = eas[0]
    idx = idxs[0]
    ptr = ptrs[0]
    src = ei[0]
    dst = ei[1]
    n = x.shape[0]

    is_loop = (src == dst).astype(jnp.float32)
    cnt = jnp.zeros((n,), jnp.float32).at[src].add(is_loop)
    indeg = jnp.zeros((n,), jnp.float32).at[dst].add(1.0)
    loopw = (cnt == 0).astype(jnp.float32)
    deg = indeg + loopw
    dinv = jax.lax.rsqrt(deg)
    diag = loopw * dinv * dinv

    def prop(h):
        hs = dinv[:, None] * h
        sacc = jnp.zeros_like(h).at[dst].add(hs[src])
        return dinv[:, None] * sacc + diag[:, None] * h

    h1 = x @ W1
    z1 = prop(h1) + b1
    h2 = z1 @ W2
    z2 = jax.nn.relu(prop(h2) + b2)
    h3 = z2 @ W3
    z3 = jax.nn.relu(prop(h3) + b3)

    h8 = jax.nn.relu(ea @ A1 + a1)
    xs = z3[src]
    t = (h8[:, :, None] * xs[:, None, :]).reshape(E, 8 * H)
    msg = t @ A2.reshape(8 * H, H) + xs @ a2.reshape(H, H)
    s4 = jnp.zeros((n, H), jnp.float32).at[dst].add(msg)
    aggr = s4 / jnp.maximum(indeg, 1.0)[:, None]
    z4 = jnp.tanh(z3 @ root + aggr + bc4)

    gi = z4 @ Wi.T + bi
    gi = jnp.concatenate(
        [gi, jnp.zeros((C * K - n, 3 * H), jnp.float32)], axis=0)
    A_ck = gi.reshape(C, K, 3 * H)

    def warm_body(s, h):
        return _gru_step(h, A_ck[:, s, :], Wh, bh)

    hw = jax.lax.fori_loop(0, K, warm_body, jnp.zeros((C, H), jnp.float32))
    h0 = jnp.concatenate([jnp.zeros((1, H), jnp.float32), hw[:-1]], axis=0)

    def main_body(h, s):
        hn = _gru_step(h, A_ck[:, s, :], Wh, bh)
        return hn, hn

    _, outs = jax.lax.scan(main_body, h0, jnp.arange(K))
    h_all = outs.transpose(1, 0, 2).reshape(C * K, H)[:n]

    lo = h_all @ Wl + bl

    deg_s = idx[1:] - idx[:-1]
    r = jax.random.randint(jax.random.fold_in(jax.random.key(42), 0),
                           (n, S), 0, 1 << 30)
    off = r % jnp.maximum(deg_s, 1)[:, None]
    nb = ptr[idx[:-1][:, None] + off]
    agg = lo[nb].sum(axis=1)
    zagg = jnp.where((deg_s > 0)[:, None], (agg + lo) / (S + 1.0), lo)
    out0 = jax.nn.softmax(zagg @ Wd + bd, axis=1)
    return jnp.stack([out0])


# speedup vs baseline: 7.2977x; 1.8025x over previous
import functools

import jax
import jax.numpy as jnp
from jax import lax
from jax.experimental import pallas as pl
from jax.experimental.pallas import tpu as pltpu
from jax.experimental.pallas import tpu_sc as plsc

N = 10000
E = 160000
H = 32
S = 5
C = 128
K = 79

NW = 32
CHK = 128
NCH = 40
EPW = CHK * NCH
EPAD = NW * EPW
DUMP = N
NACC = 10112

_MESH = plsc.VectorSubcoreMesh(core_axis_name="c", subcore_axis_name="s")

NGRP = 5
NBUF = 8


@functools.partial(
    pl.kernel,
    out_type=jax.ShapeDtypeStruct((2, N, H), jnp.float32),
    mesh=_MESH,
    compiler_params=pltpu.CompilerParams(use_tc_tiling_on_sc=False),
    scratch_types=[
        pltpu.VMEM((NCH, CHK), jnp.int32),
        pltpu.VMEM((NCH, CHK), jnp.int32),
        pltpu.VMEM((NBUF, CHK, H), jnp.float32),
        pltpu.VMEM((632, H), jnp.float32),
        pltpu.VMEM_SHARED((NACC, H), jnp.float32),
        pltpu.SemaphoreType.DMA,
        pltpu.SemaphoreType.DMA,
    ],
)
def _prop_sc(hs_hbm, src_hbm, dst_hbm, zer_hbm, out_hbm,
             sidx, didx, rows, stage, acc, gsem, ssem):
    c = lax.axis_index("c")
    s = lax.axis_index("s")
    w = s * 2 + c
    pltpu.sync_copy(zer_hbm.at[pl.ds(s * 632, 632)], stage)
    pltpu.sync_copy(stage, acc.at[pl.ds(s * 632, 632)])
    pltpu.sync_copy(src_hbm.at[w], sidx)
    pltpu.sync_copy(dst_hbm.at[w], didx)
    plsc.subcore_barrier()

    def grp(g, carry):
        h = []
        for b in range(NBUF):
            j = g * NBUF + b
            h.append(pltpu.async_copy(hs_hbm.at[sidx.at[j]], rows.at[b], gsem))
        for hd in h:
            hd.wait()
        sc = []
        for b in range(NBUF):
            j = g * NBUF + b
            sc.append(pltpu.async_copy(rows.at[b], acc.at[didx.at[j]], ssem,
                                       add=True))
        for hd in sc:
            hd.wait()
        return carry

    lax.fori_loop(0, NGRP, grp, 0)
    plsc.subcore_barrier()

    @pl.when(s < 15)
    def _():
        pltpu.sync_copy(acc.at[pl.ds(s * 632, 632)], stage)
        pltpu.sync_copy(stage, out_hbm.at[c, pl.ds(s * 632, 632)])

    @pl.when(s == 15)
    def _():
        pltpu.sync_copy(acc.at[pl.ds(9480, 520)], stage.at[pl.ds(0, 520)])
        pltpu.sync_copy(stage.at[pl.ds(0, 520)], out_hbm.at[c, pl.ds(9480, 520)])


def _gru_step(h, g, Wh, bh):
    gh = h @ Wh.T + bh
    ir, iz, i_n = jnp.split(g, 3, axis=1)
    hr, hz, hn = jnp.split(gh, 3, axis=1)
    r = jax.nn.sigmoid(ir + hr)
    zg = jax.nn.sigmoid(iz + hz)
    ng = jnp.tanh(i_n + r * hn)
    return (1.0 - zg) * ng + zg * h


def kernel(x, eis, eas, idxs, ptrs, W1, b1, W2, b2, W3, b3, A1, a1, A2, a2,
           root, bc4, Wi, bi, Wh, bh, Wl, bl, Wd, bd):
    ei = eis[0]
    ea = eas[0]
    idx = idxs[0]
    ptr = ptrs[0]
    src = ei[0]
    dst = ei[1]
    n = x.shape[0]

    is_loop = (src == dst).astype(jnp.float32)
    cnt = jnp.zeros((n,), jnp.float32).at[src].add(is_loop)
    indeg = jnp.zeros((n,), jnp.float32).at[dst].add(1.0)
    loopw = (cnt == 0).astype(jnp.float32)
    deg = indeg + loopw
    dinv = jax.lax.rsqrt(deg)
    diag = loopw * dinv * dinv

    pad = EPAD - E
    src_p = jnp.concatenate([src, jnp.zeros((pad,), src.dtype)])
    dst_p = jnp.concatenate([dst, jnp.full((pad,), DUMP, dst.dtype)])
    src3 = src_p.reshape(NW, NCH, CHK).astype(jnp.int32)
    dst3 = dst_p.reshape(NW, NCH, CHK).astype(jnp.int32)
    zer = jnp.zeros((NACC, H), jnp.float32)

    def prop(h):
        hs = dinv[:, None] * h
        p = _prop_sc(hs, src3, dst3, zer)
        return dinv[:, None] * (p[0] + p[1]) + diag[:, None] * h

    h1 = x @ W1
    z1 = prop(h1) + b1
    h2 = z1 @ W2
    z2 = jax.nn.relu(prop(h2) + b2)
    h3 = z2 @ W3
    z3 = jax.nn.relu(prop(h3) + b3)

    h8 = jax.nn.relu(ea @ A1 + a1)
    xs = z3[src]
    t = (h8[:, :, None] * xs[:, None, :]).reshape(E, 8 * H)
    msg = t @ A2.reshape(8 * H, H) + xs @ a2.reshape(H, H)
    s4 = jnp.zeros((n, H), jnp.float32).at[dst].add(msg)
    aggr = s4 / jnp.maximum(indeg, 1.0)[:, None]
    z4 = jnp.tanh(z3 @ root + aggr + bc4)

    gi = z4 @ Wi.T + bi
    gi = jnp.concatenate(
        [gi, jnp.zeros((C * K - n, 3 * H), jnp.float32)], axis=0)
    A_ck = gi.reshape(C, K, 3 * H)

    def warm_body(s_, h):
        return _gru_step(h, A_ck[:, s_, :], Wh, bh)

    hw = lax.fori_loop(0, K, warm_body, jnp.zeros((C, H), jnp.float32))
    h0 = jnp.concatenate([jnp.zeros((1, H), jnp.float32), hw[:-1]], axis=0)

    def main_body(h, s_):
        hn = _gru_step(h, A_ck[:, s_, :], Wh, bh)
        return hn, hn

    _, outs = lax.scan(main_body, h0, jnp.arange(K))
    h_all = outs.transpose(1, 0, 2).reshape(C * K, H)[:n]

    lo = h_all @ Wl + bl

    deg_s = idx[1:] - idx[:-1]
    r = jax.random.randint(jax.random.fold_in(jax.random.key(42), 0),
                           (n, S), 0, 1 << 30)
    off = r % jnp.maximum(deg_s, 1)[:, None]
    nb = ptr[idx[:-1][:, None] + off]
    agg = lo[nb].sum(axis=1)
    zagg = jnp.where((deg_s > 0)[:, None], (agg + lo) / (S + 1.0), lo)
    out0 = jax.nn.softmax(zagg @ Wd + bd, axis=1)
    return jnp.stack([out0])


# speedup vs baseline: 9.3833x; 1.2858x over previous
import functools

import jax
import jax.numpy as jnp
from jax import lax
from jax.experimental import pallas as pl
from jax.experimental.pallas import tpu as pltpu
from jax.experimental.pallas import tpu_sc as plsc

N = 10000
E = 160000
H = 32
S = 5
C = 128
K = 79

NW = 32
CHK = 128
NCH = 40
EPW = CHK * NCH
EPAD = NW * EPW
DUMP = N
NACC = 10112

_MESH = plsc.VectorSubcoreMesh(core_axis_name="c", subcore_axis_name="s")

NGRP = 5
NBUF = 8


@functools.partial(
    pl.kernel,
    out_type=jax.ShapeDtypeStruct((2, N, H), jnp.float32),
    mesh=_MESH,
    compiler_params=pltpu.CompilerParams(use_tc_tiling_on_sc=False),
    scratch_types=[
        pltpu.VMEM((NCH, CHK), jnp.int32),
        pltpu.VMEM((NCH, CHK), jnp.int32),
        pltpu.VMEM((NBUF, CHK, H), jnp.float32),
        pltpu.VMEM((632, H), jnp.float32),
        pltpu.VMEM_SHARED((NACC, H), jnp.float32),
        pltpu.SemaphoreType.DMA,
        pltpu.SemaphoreType.DMA,
    ],
)
def _prop_sc(hs_hbm, src_hbm, dst_hbm, zer_hbm, out_hbm,
             sidx, didx, rows, stage, acc, gsem, ssem):
    c = lax.axis_index("c")
    s = lax.axis_index("s")
    w = s * 2 + c
    pltpu.sync_copy(zer_hbm.at[pl.ds(s * 632, 632)], stage)
    pltpu.sync_copy(stage, acc.at[pl.ds(s * 632, 632)])
    pltpu.sync_copy(src_hbm.at[w], sidx)
    pltpu.sync_copy(dst_hbm.at[w], didx)
    plsc.subcore_barrier()

    def grp(g, carry):
        h = []
        for b in range(NBUF):
            j = g * NBUF + b
            h.append(pltpu.async_copy(hs_hbm.at[sidx.at[j]], rows.at[b], gsem))
        for hd in h:
            hd.wait()
        sc = []
        for b in range(NBUF):
            j = g * NBUF + b
            sc.append(pltpu.async_copy(rows.at[b], acc.at[didx.at[j]], ssem,
                                       add=True))
        for hd in sc:
            hd.wait()
        return carry

    lax.fori_loop(0, NGRP, grp, 0)
    plsc.subcore_barrier()

    @pl.when(s < 15)
    def _():
        pltpu.sync_copy(acc.at[pl.ds(s * 632, 632)], stage)
        pltpu.sync_copy(stage, out_hbm.at[c, pl.ds(s * 632, 632)])

    @pl.when(s == 15)
    def _():
        pltpu.sync_copy(acc.at[pl.ds(9480, 520)], stage.at[pl.ds(0, 520)])
        pltpu.sync_copy(stage.at[pl.ds(0, 520)], out_hbm.at[c, pl.ds(9480, 520)])


@functools.partial(
    pl.kernel,
    out_type=jax.ShapeDtypeStruct((EPAD, H), jnp.float32),
    mesh=_MESH,
    compiler_params=pltpu.CompilerParams(use_tc_tiling_on_sc=False),
    scratch_types=[
        pltpu.VMEM((NCH, CHK), jnp.int32),
        pltpu.VMEM((NBUF, CHK, H), jnp.float32),
        pltpu.SemaphoreType.DMA,
        pltpu.SemaphoreType.DMA,
    ],
)
def _gath_sc(h_hbm, src_hbm, out_hbm, sidx, rows, gsem, wsem):
    c = lax.axis_index("c")
    s = lax.axis_index("s")
    w = s * 2 + c
    pltpu.sync_copy(src_hbm.at[w], sidx)

    def grp(g, carry):
        hnd = []
        for b in range(NBUF):
            j = g * NBUF + b
            hnd.append(pltpu.async_copy(h_hbm.at[sidx.at[j]], rows.at[b], gsem))
        for hd in hnd:
            hd.wait()
        wh = []
        for b in range(NBUF):
            j = g * NBUF + b
            wh.append(pltpu.async_copy(
                rows.at[b], out_hbm.at[pl.ds(w * EPW + j * CHK, CHK)], wsem))
        for hd in wh:
            hd.wait()
        return carry

    lax.fori_loop(0, NGRP, grp, 0)


@functools.partial(
    pl.kernel,
    out_type=jax.ShapeDtypeStruct((2, N, H), jnp.float32),
    mesh=_MESH,
    compiler_params=pltpu.CompilerParams(use_tc_tiling_on_sc=False),
    scratch_types=[
        pltpu.VMEM((NCH, CHK), jnp.int32),
        pltpu.VMEM((NBUF, CHK, H), jnp.float32),
        pltpu.VMEM((632, H), jnp.float32),
        pltpu.VMEM_SHARED((NACC, H), jnp.float32),
        pltpu.SemaphoreType.DMA,
        pltpu.SemaphoreType.DMA,
    ],
)
def _scat_sc(rows_hbm, dst_hbm, zer_hbm, out_hbm,
             didx, rows, stage, acc, gsem, ssem):
    c = lax.axis_index("c")
    s = lax.axis_index("s")
    w = s * 2 + c
    pltpu.sync_copy(zer_hbm.at[pl.ds(s * 632, 632)], stage)
    pltpu.sync_copy(stage, acc.at[pl.ds(s * 632, 632)])
    pltpu.sync_copy(dst_hbm.at[w], didx)
    plsc.subcore_barrier()

    def grp(g, carry):
        hnd = []
        for b in range(NBUF):
            j = g * NBUF + b
            hnd.append(pltpu.async_copy(
                rows_hbm.at[pl.ds(w * EPW + j * CHK, CHK)], rows.at[b], gsem))
        for hd in hnd:
            hd.wait()
        sc = []
        for b in range(NBUF):
            j = g * NBUF + b
            sc.append(pltpu.async_copy(rows.at[b], acc.at[didx.at[j]], ssem,
                                       add=True))
        for hd in sc:
            hd.wait()
        return carry

    lax.fori_loop(0, NGRP, grp, 0)
    plsc.subcore_barrier()

    @pl.when(s < 15)
    def _():
        pltpu.sync_copy(acc.at[pl.ds(s * 632, 632)], stage)
        pltpu.sync_copy(stage, out_hbm.at[c, pl.ds(s * 632, 632)])

    @pl.when(s == 15)
    def _():
        pltpu.sync_copy(acc.at[pl.ds(9480, 520)], stage.at[pl.ds(0, 520)])
        pltpu.sync_copy(stage.at[pl.ds(0, 520)], out_hbm.at[c, pl.ds(9480, 520)])


@functools.partial(
    pl.kernel,
    out_type=(jax.ShapeDtypeStruct((NW, NACC), jnp.float32),
              jax.ShapeDtypeStruct((NW, NACC), jnp.float32)),
    mesh=_MESH,
    compiler_params=pltpu.CompilerParams(use_tc_tiling_on_sc=False),
    scratch_types=[
        pltpu.VMEM((NCH, CHK), jnp.int32),
        pltpu.VMEM((NCH, CHK), jnp.int32),
        pltpu.VMEM((NACC,), jnp.float32),
        pltpu.VMEM((NACC,), jnp.float32),
    ],
)
def _hist_sc(src_hbm, dst_hbm, ideg_hbm, cnt_hbm, sidx, didx, hdeg, hcnt):
    c = lax.axis_index("c")
    s = lax.axis_index("s")
    w = s * 2 + c
    pltpu.sync_copy(src_hbm.at[w], sidx)
    pltpu.sync_copy(dst_hbm.at[w], didx)
    z16 = jnp.zeros((16,), jnp.float32)

    def zbody(i, carry):
        hdeg[pl.ds(i * 16, 16)] = z16
        hcnt[pl.ds(i * 16, 16)] = z16
        return carry

    lax.fori_loop(0, NACC // 16, zbody, 0)
    ones = jnp.ones((16,), jnp.float32)

    def ebody(j, carry):
        for k in range(CHK // 16):
            d16 = didx[j, pl.ds(k * 16, 16)]
            s16 = sidx[j, pl.ds(k * 16, 16)]
            plsc.addupdate_scatter(hdeg, [d16], ones)
            plsc.addupdate_scatter(hcnt, [s16], ones, mask=s16 == d16)
        return carry

    lax.fori_loop(0, NCH, ebody, 0)
    pltpu.sync_copy(hdeg, ideg_hbm.at[w])
    pltpu.sync_copy(hcnt, cnt_hbm.at[w])


BLK = 2048


def _msg_body(ea_ref, xs_ref, A1_ref, a1_ref, A2_ref, a2r_ref, out_ref):
    h8 = jnp.maximum(
        jnp.dot(ea_ref[...], A1_ref[...],
                preferred_element_type=jnp.float32) + a1_ref[...], 0.0)
    xs = xs_ref[...]
    acc = jnp.dot(xs, a2r_ref[...], preferred_element_type=jnp.float32)
    for j in range(8):
        acc += h8[:, j:j + 1] * jnp.dot(xs, A2_ref[j],
                                        preferred_element_type=jnp.float32)
    out_ref[...] = acc


def _msg_tc(ea_pad, xs, A1, a1, A2r3, a2r):
    return pl.pallas_call(
        _msg_body,
        grid=(EPAD // BLK,),
        in_specs=[
            pl.BlockSpec((BLK, 4), lambda i: (i, 0)),
            pl.BlockSpec((BLK, H), lambda i: (i, 0)),
            pl.BlockSpec((4, 8), lambda i: (0, 0)),
            pl.BlockSpec((1, 8), lambda i: (0, 0)),
            pl.BlockSpec((8, H, H), lambda i: (0, 0, 0)),
            pl.BlockSpec((H, H), lambda i: (0, 0)),
        ],
        out_specs=pl.BlockSpec((BLK, H), lambda i: (i, 0)),
        out_shape=jax.ShapeDtypeStruct((EPAD, H), jnp.float32),
    )(ea_pad, xs, A1, a1, A2r3, a2r)


def _gru_step(h, g, Wh, bh):
    gh = h @ Wh.T + bh
    ir, iz, i_n = jnp.split(g, 3, axis=1)
    hr, hz, hn = jnp.split(gh, 3, axis=1)
    r = jax.nn.sigmoid(ir + hr)
    zg = jax.nn.sigmoid(iz + hz)
    ng = jnp.tanh(i_n + r * hn)
    return (1.0 - zg) * ng + zg * h


def kernel(x, eis, eas, idxs, ptrs, W1, b1, W2, b2, W3, b3, A1, a1, A2, a2,
           root, bc4, Wi, bi, Wh, bh, Wl, bl, Wd, bd):
    ei = eis[0]
    ea = eas[0]
    idx = idxs[0]
    ptr = ptrs[0]
    src = ei[0]
    dst = ei[1]
    n = x.shape[0]

    pad = EPAD - E
    src_p = jnp.concatenate([src, jnp.zeros((pad,), src.dtype)])
    dst_p = jnp.concatenate([dst, jnp.full((pad,), DUMP, dst.dtype)])
    src3 = src_p.reshape(NW, NCH, CHK).astype(jnp.int32)
    dst3 = dst_p.reshape(NW, NCH, CHK).astype(jnp.int32)
    zer = jnp.zeros((NACC, H), jnp.float32)

    is_loop = (src == dst).astype(jnp.float32)
    cnt = jnp.zeros((n,), jnp.float32).at[src].add(is_loop)
    indeg = jnp.zeros((n,), jnp.float32).at[dst].add(1.0)
    loopw = (cnt == 0).astype(jnp.float32)
    deg = indeg + loopw
    dinv = jax.lax.rsqrt(deg)
    diag = loopw * dinv * dinv

    def prop(h):
        hs = dinv[:, None] * h
        p = _prop_sc(hs, src3, dst3, zer)
        return dinv[:, None] * (p[0] + p[1]) + diag[:, None] * h

    h1 = x @ W1
    z1 = prop(h1) + b1
    h2 = z1 @ W2
    z2 = jax.nn.relu(prop(h2) + b2)
    h3 = z2 @ W3
    z3 = jax.nn.relu(prop(h3) + b3)

    ea_pad = jnp.concatenate([ea, jnp.zeros((pad, ea.shape[1]), ea.dtype)])
    xs_pad = _gath_sc(z3, src3)
    msg_pad = _msg_tc(ea_pad, xs_pad, A1, a1.reshape(1, 8),
                      A2.reshape(8, H, H), a2.reshape(H, H))
    s4p = _scat_sc(msg_pad, dst3, zer)
    s4 = s4p[0] + s4p[1]
    aggr = s4 / jnp.maximum(indeg, 1.0)[:, None]
    z4 = jnp.tanh(z3 @ root + aggr + bc4)

    gi = z4 @ Wi.T + bi
    gi = jnp.concatenate(
        [gi, jnp.zeros((C * K - n, 3 * H), jnp.float32)], axis=0)
    A_ck = gi.reshape(C, K, 3 * H)

    def warm_body(s_, h):
        return _gru_step(h, A_ck[:, s_, :], Wh, bh)

    hw = lax.fori_loop(0, K, warm_body, jnp.zeros((C, H), jnp.float32))
    h0 = jnp.concatenate([jnp.zeros((1, H), jnp.float32), hw[:-1]], axis=0)

    def main_body(h, s_):
        hn = _gru_step(h, A_ck[:, s_, :], Wh, bh)
        return hn, hn

    _, outs = lax.scan(main_body, h0, jnp.arange(K))
    h_all = outs.transpose(1, 0, 2).reshape(C * K, H)[:n]

    lo = h_all @ Wl + bl

    deg_s = idx[1:] - idx[:-1]
    r = jax.random.randint(jax.random.fold_in(jax.random.key(42), 0),
                           (n, S), 0, 1 << 30)
    off = r % jnp.maximum(deg_s, 1)[:, None]
    nb = ptr[idx[:-1][:, None] + off]
    agg = lo[nb].sum(axis=1)
    zagg = jnp.where((deg_s > 0)[:, None], (agg + lo) / (S + 1.0), lo)
    out0 = jax.nn.softmax(zagg @ Wd + bd, axis=1)
    return jnp.stack([out0])


# speedup vs baseline: 11.1871x; 1.1922x over previous
import functools

import jax
import jax.numpy as jnp
from jax import lax
from jax.experimental import pallas as pl
from jax.experimental.pallas import tpu as pltpu
from jax.experimental.pallas import tpu_sc as plsc

N = 10000
E = 160000
H = 32
S = 5
C = 128
K = 79

NW = 32
CHK = 128
NCH = 40
EPW = CHK * NCH
EPAD = NW * EPW
DUMP = N
NACC = 10112

_MESH = plsc.VectorSubcoreMesh(core_axis_name="c", subcore_axis_name="s")

NGRP = 5
NBUF = 8


@functools.partial(
    pl.kernel,
    out_type=jax.ShapeDtypeStruct((2, N, H), jnp.float32),
    mesh=_MESH,
    compiler_params=pltpu.CompilerParams(use_tc_tiling_on_sc=False),
    scratch_types=[
        pltpu.VMEM((NCH, CHK), jnp.int32),
        pltpu.VMEM((NCH, CHK), jnp.int32),
        pltpu.VMEM((NBUF, CHK, H), jnp.float32),
        pltpu.VMEM((632, H), jnp.float32),
        pltpu.VMEM_SHARED((NACC, H), jnp.float32),
        pltpu.SemaphoreType.DMA,
        pltpu.SemaphoreType.DMA,
    ],
)
def _prop_sc(hs_hbm, src_hbm, dst_hbm, zer_hbm, out_hbm,
             sidx, didx, rows, stage, acc, gsem, ssem):
    c = lax.axis_index("c")
    s = lax.axis_index("s")
    w = s * 2 + c
    pltpu.sync_copy(zer_hbm.at[pl.ds(s * 632, 632)], stage)
    pltpu.sync_copy(stage, acc.at[pl.ds(s * 632, 632)])
    pltpu.sync_copy(src_hbm.at[w], sidx)
    pltpu.sync_copy(dst_hbm.at[w], didx)
    plsc.subcore_barrier()

    def grp(g, carry):
        h = []
        for b in range(NBUF):
            j = g * NBUF + b
            h.append(pltpu.async_copy(hs_hbm.at[sidx.at[j]], rows.at[b], gsem))
        for hd in h:
            hd.wait()
        sc = []
        for b in range(NBUF):
            j = g * NBUF + b
            sc.append(pltpu.async_copy(rows.at[b], acc.at[didx.at[j]], ssem,
                                       add=True))
        for hd in sc:
            hd.wait()
        return carry

    lax.fori_loop(0, NGRP, grp, 0)
    plsc.subcore_barrier()

    @pl.when(s < 15)
    def _():
        pltpu.sync_copy(acc.at[pl.ds(s * 632, 632)], stage)
        pltpu.sync_copy(stage, out_hbm.at[c, pl.ds(s * 632, 632)])

    @pl.when(s == 15)
    def _():
        pltpu.sync_copy(acc.at[pl.ds(9480, 520)], stage.at[pl.ds(0, 520)])
        pltpu.sync_copy(stage.at[pl.ds(0, 520)], out_hbm.at[c, pl.ds(9480, 520)])


@functools.partial(
    pl.kernel,
    out_type=jax.ShapeDtypeStruct((EPAD, H), jnp.float32),
    mesh=_MESH,
    compiler_params=pltpu.CompilerParams(use_tc_tiling_on_sc=False),
    scratch_types=[
        pltpu.VMEM((NCH, CHK), jnp.int32),
        pltpu.VMEM((NBUF, CHK, H), jnp.float32),
        pltpu.SemaphoreType.DMA,
        pltpu.SemaphoreType.DMA,
    ],
)
def _gath_sc(h_hbm, src_hbm, out_hbm, sidx, rows, gsem, wsem):
    c = lax.axis_index("c")
    s = lax.axis_index("s")
    w = s * 2 + c
    pltpu.sync_copy(src_hbm.at[w], sidx)

    def grp(g, carry):
        hnd = []
        for b in range(NBUF):
            j = g * NBUF + b
            hnd.append(pltpu.async_copy(h_hbm.at[sidx.at[j]], rows.at[b], gsem))
        for hd in hnd:
            hd.wait()
        wh = []
        for b in range(NBUF):
            j = g * NBUF + b
            wh.append(pltpu.async_copy(
                rows.at[b], out_hbm.at[pl.ds(w * EPW + j * CHK, CHK)], wsem))
        for hd in wh:
            hd.wait()
        return carry

    lax.fori_loop(0, NGRP, grp, 0)


@functools.partial(
    pl.kernel,
    out_type=jax.ShapeDtypeStruct((2, N, H), jnp.float32),
    mesh=_MESH,
    compiler_params=pltpu.CompilerParams(use_tc_tiling_on_sc=False),
    scratch_types=[
        pltpu.VMEM((NCH, CHK), jnp.int32),
        pltpu.VMEM((NBUF, CHK, H), jnp.float32),
        pltpu.VMEM((632, H), jnp.float32),
        pltpu.VMEM_SHARED((NACC, H), jnp.float32),
        pltpu.SemaphoreType.DMA,
        pltpu.SemaphoreType.DMA,
    ],
)
def _scat_sc(rows_hbm, dst_hbm, zer_hbm, out_hbm,
             didx, rows, stage, acc, gsem, ssem):
    c = lax.axis_index("c")
    s = lax.axis_index("s")
    w = s * 2 + c
    pltpu.sync_copy(zer_hbm.at[pl.ds(s * 632, 632)], stage)
    pltpu.sync_copy(stage, acc.at[pl.ds(s * 632, 632)])
    pltpu.sync_copy(dst_hbm.at[w], didx)
    plsc.subcore_barrier()

    def grp(g, carry):
        hnd = []
        for b in range(NBUF):
            j = g * NBUF + b
            hnd.append(pltpu.async_copy(
                rows_hbm.at[pl.ds(w * EPW + j * CHK, CHK)], rows.at[b], gsem))
        for hd in hnd:
            hd.wait()
        sc = []
        for b in range(NBUF):
            j = g * NBUF + b
            sc.append(pltpu.async_copy(rows.at[b], acc.at[didx.at[j]], ssem,
                                       add=True))
        for hd in sc:
            hd.wait()
        return carry

    lax.fori_loop(0, NGRP, grp, 0)
    plsc.subcore_barrier()

    @pl.when(s < 15)
    def _():
        pltpu.sync_copy(acc.at[pl.ds(s * 632, 632)], stage)
        pltpu.sync_copy(stage, out_hbm.at[c, pl.ds(s * 632, 632)])

    @pl.when(s == 15)
    def _():
        pltpu.sync_copy(acc.at[pl.ds(9480, 520)], stage.at[pl.ds(0, 520)])
        pltpu.sync_copy(stage.at[pl.ds(0, 520)], out_hbm.at[c, pl.ds(9480, 520)])


@functools.partial(
    pl.kernel,
    out_type=(jax.ShapeDtypeStruct((NW, NACC), jnp.float32),
              jax.ShapeDtypeStruct((NW, NACC), jnp.float32)),
    mesh=_MESH,
    compiler_params=pltpu.CompilerParams(use_tc_tiling_on_sc=False),
    scratch_types=[
        pltpu.VMEM((NCH, CHK), jnp.int32),
        pltpu.VMEM((NCH, CHK), jnp.int32),
        pltpu.VMEM((NACC,), jnp.float32),
        pltpu.VMEM((NACC,), jnp.float32),
    ],
)
def _hist_sc(src_hbm, dst_hbm, ideg_hbm, cnt_hbm, sidx, didx, hdeg, hcnt):
    c = lax.axis_index("c")
    s = lax.axis_index("s")
    w = s * 2 + c
    pltpu.sync_copy(src_hbm.at[w], sidx)
    pltpu.sync_copy(dst_hbm.at[w], didx)
    z16 = jnp.zeros((16,), jnp.float32)

    def zbody(i, carry):
        hdeg[pl.ds(i * 16, 16)] = z16
        hcnt[pl.ds(i * 16, 16)] = z16
        return carry

    lax.fori_loop(0, NACC // 16, zbody, 0)
    ones = jnp.ones((16,), jnp.float32)

    def ebody(j, carry):
        for k in range(CHK // 16):
            d16 = didx[j, pl.ds(k * 16, 16)]
            s16 = sidx[j, pl.ds(k * 16, 16)]
            plsc.addupdate_scatter(hdeg, [d16], ones)
            plsc.addupdate_scatter(hcnt, [s16], ones, mask=s16 == d16)
        return carry

    lax.fori_loop(0, NCH, ebody, 0)
    pltpu.sync_copy(hdeg, ideg_hbm.at[w])
    pltpu.sync_copy(hcnt, cnt_hbm.at[w])


BLK = 2048


def _msg_body(ea_ref, xs_ref, A1_ref, a1_ref, A2_ref, a2r_ref, out_ref):
    h8 = jnp.maximum(
        jnp.dot(ea_ref[...], A1_ref[...],
                preferred_element_type=jnp.float32) + a1_ref[...], 0.0)
    xs = xs_ref[...]
    acc = jnp.dot(xs, a2r_ref[...], preferred_element_type=jnp.float32)
    for j in range(8):
        acc += h8[:, j:j + 1] * jnp.dot(xs, A2_ref[j],
                                        preferred_element_type=jnp.float32)
    out_ref[...] = acc


def _msg_tc(ea_pad, xs, A1, a1, A2r3, a2r):
    return pl.pallas_call(
        _msg_body,
        grid=(EPAD // BLK,),
        in_specs=[
            pl.BlockSpec((BLK, 4), lambda i: (i, 0)),
            pl.BlockSpec((BLK, H), lambda i: (i, 0)),
            pl.BlockSpec((4, 8), lambda i: (0, 0)),
            pl.BlockSpec((1, 8), lambda i: (0, 0)),
            pl.BlockSpec((8, H, H), lambda i: (0, 0, 0)),
            pl.BlockSpec((H, H), lambda i: (0, 0)),
        ],
        out_specs=pl.BlockSpec((BLK, H), lambda i: (i, 0)),
        out_shape=jax.ShapeDtypeStruct((EPAD, H), jnp.float32),
    )(ea_pad, xs, A1, a1, A2r3, a2r)


NPD = 10240
NPW = 320
DCH = 64


@functools.partial(
    pl.kernel,
    out_type=jax.ShapeDtypeStruct((NPD, H), jnp.float32),
    mesh=_MESH,
    compiler_params=pltpu.CompilerParams(use_tc_tiling_on_sc=False),
    scratch_types=[
        pltpu.VMEM((S, NPW), jnp.int32),
        pltpu.VMEM((2, DCH, H), jnp.float32),
        pltpu.SemaphoreType.DMA,
        pltpu.SemaphoreType.DMA,
    ],
)
def _dec_sc(lo_hbm, nbT_hbm, g_hbm, nbbuf, rows, sem, osem):
    c = lax.axis_index("c")
    s = lax.axis_index("s")
    w = s * 2 + c
    base = w * NPW
    pltpu.sync_copy(nbT_hbm.at[:, pl.ds(base, NPW)], nbbuf)
    ohnd = []
    for t in range(NPW // DCH):
        b = t % 2
        pltpu.async_copy(lo_hbm.at[nbbuf.at[0, pl.ds(t * DCH, DCH)]],
                         rows.at[b], sem).wait()
        hnd = [pltpu.async_copy(lo_hbm.at[nbbuf.at[k, pl.ds(t * DCH, DCH)]],
                                rows.at[b], sem, add=True)
               for k in range(1, S)]
        for hd in hnd:
            hd.wait()
        ohnd.append(pltpu.async_copy(
            rows.at[b], g_hbm.at[pl.ds(base + t * DCH, DCH)], osem))
        if len(ohnd) == 2:
            ohnd.pop(0).wait()
    for hd in ohnd:
        hd.wait()


def _gru_body(gi_ref, WhT_ref, bh_ref, Wl_ref, bl_ref, lo_ref):
    WhT = WhT_ref[...]
    bh2 = bh_ref[...]
    Wl = Wl_ref[...]
    bl2 = bl_ref[...]

    def step(h, g):
        gh = jnp.dot(h, WhT, preferred_element_type=jnp.float32) + bh2
        r = jax.nn.sigmoid(g[:, :H] + gh[:, :H])
        zg = jax.nn.sigmoid(g[:, H:2 * H] + gh[:, H:2 * H])
        ng = jnp.tanh(g[:, 2 * H:] + r * gh[:, 2 * H:])
        return (1.0 - zg) * ng + zg * h

    def warm(s_, h):
        return step(h, gi_ref[s_])

    h = lax.fori_loop(0, K, warm, jnp.zeros((C, H), jnp.float32))
    h = jnp.concatenate([jnp.zeros((1, H), jnp.float32), h[:C - 1, :]], axis=0)

    def main(s_, h):
        h = step(h, gi_ref[s_])
        lo_ref[s_] = jnp.dot(h, Wl, preferred_element_type=jnp.float32) + bl2
        return h

    lax.fori_loop(0, K, main, h)


def _gru_tc(giT, WhT, bh2, Wl, bl2):
    return pl.pallas_call(
        _gru_body,
        out_shape=jax.ShapeDtypeStruct((K, C, H), jnp.float32),
    )(giT, WhT, bh2, Wl, bl2)


def kernel(x, eis, eas, idxs, ptrs, W1, b1, W2, b2, W3, b3, A1, a1, A2, a2,
           root, bc4, Wi, bi, Wh, bh, Wl, bl, Wd, bd):
    ei = eis[0]
    ea = eas[0]
    idx = idxs[0]
    ptr = ptrs[0]
    src = ei[0]
    dst = ei[1]
    n = x.shape[0]

    pad = EPAD - E
    src_p = jnp.concatenate([src, jnp.zeros((pad,), src.dtype)])
    dst_p = jnp.concatenate([dst, jnp.full((pad,), DUMP, dst.dtype)])
    src3 = src_p.reshape(NW, NCH, CHK).astype(jnp.int32)
    dst3 = dst_p.reshape(NW, NCH, CHK).astype(jnp.int32)
    zer = jnp.zeros((NACC, H), jnp.float32)

    is_loop = (src == dst).astype(jnp.float32)
    cnt = jnp.zeros((n,), jnp.float32).at[src].add(is_loop)
    indeg = jnp.zeros((n,), jnp.float32).at[dst].add(1.0)
    loopw = (cnt == 0).astype(jnp.float32)
    deg = indeg + loopw
    dinv = jax.lax.rsqrt(deg)
    diag = loopw * dinv * dinv

    def prop(h):
        hs = dinv[:, None] * h
        p = _prop_sc(hs, src3, dst3, zer)
        return dinv[:, None] * (p[0] + p[1]) + diag[:, None] * h

    h1 = x @ W1
    z1 = prop(h1) + b1
    h2 = z1 @ W2
    z2 = jax.nn.relu(prop(h2) + b2)
    h3 = z2 @ W3
    z3 = jax.nn.relu(prop(h3) + b3)

    ea_pad = jnp.concatenate([ea, jnp.zeros((pad, ea.shape[1]), ea.dtype)])
    xs_pad = _gath_sc(z3, src3)
    msg_pad = _msg_tc(ea_pad, xs_pad, A1, a1.reshape(1, 8),
                      A2.reshape(8, H, H), a2.reshape(H, H))
    s4p = _scat_sc(msg_pad, dst3, zer)
    s4 = s4p[0] + s4p[1]
    aggr = s4 / jnp.maximum(indeg, 1.0)[:, None]
    z4 = jnp.tanh(z3 @ root + aggr + bc4)

    gi = z4 @ Wi.T + bi
    gi = jnp.concatenate(
        [gi, jnp.zeros((C * K - n, 3 * H), jnp.float32)], axis=0)
    giT = gi.reshape(C, K, 3 * H).transpose(1, 0, 2)
    loT = _gru_tc(giT, Wh.T, bh.reshape(1, 3 * H), Wl, bl.reshape(1, H))
    lo_pad = loT.transpose(1, 0, 2).reshape(C * K, H)
    lo = lo_pad[:n]

    deg_s = idx[1:] - idx[:-1]
    r = jax.random.randint(jax.random.fold_in(jax.random.key(42), 0),
                           (n, S), 0, 1 << 30)
    off = r % jnp.maximum(deg_s, 1)[:, None]
    nb = ptr[idx[:-1][:, None] + off]
    nbT = jnp.concatenate(
        [nb.astype(jnp.int32).T, jnp.zeros((S, NPD - n), jnp.int32)], axis=1)
    agg = _dec_sc(lo_pad, nbT)[:n]
    zagg = jnp.where((deg_s > 0)[:, None], (agg + lo) / (S + 1.0), lo)
    out0 = jax.nn.softmax(zagg @ Wd + bd, axis=1)
    return jnp.stack([out0])


# speedup vs baseline: 13.5990x; 1.2156x over previous
import functools

import jax
import jax.numpy as jnp
from jax import lax
from jax.experimental import pallas as pl
from jax.experimental.pallas import tpu as pltpu
from jax.experimental.pallas import tpu_sc as plsc

N = 10000
E = 160000
H = 32
S = 5
C = 128
K = 79

NW = 32
CHK = 128
NCH = 40
EPW = CHK * NCH
EPAD = NW * EPW
DUMP = N
NACC = 10112

_MESH = plsc.VectorSubcoreMesh(core_axis_name="c", subcore_axis_name="s")

NGRP = 5
NBUF = 8


@functools.partial(
    pl.kernel,
    out_type=jax.ShapeDtypeStruct((2, N, H), jnp.float32),
    mesh=_MESH,
    compiler_params=pltpu.CompilerParams(use_tc_tiling_on_sc=False),
    scratch_types=[
        pltpu.VMEM((NCH, CHK), jnp.int32),
        pltpu.VMEM((NCH, CHK), jnp.int32),
        pltpu.VMEM((NBUF, CHK, H), jnp.float32),
        pltpu.VMEM((632, H), jnp.float32),
        pltpu.VMEM_SHARED((NACC, H), jnp.float32),
        pltpu.SemaphoreType.DMA,
        pltpu.SemaphoreType.DMA,
    ],
)
def _prop_sc(hs_hbm, src_hbm, dst_hbm, zer_hbm, out_hbm,
             sidx, didx, rows, stage, acc, gsem, ssem):
    c = lax.axis_index("c")
    s = lax.axis_index("s")
    w = s * 2 + c
    pltpu.sync_copy(zer_hbm.at[pl.ds(s * 632, 632)], stage)
    pltpu.sync_copy(stage, acc.at[pl.ds(s * 632, 632)])
    pltpu.sync_copy(src_hbm.at[w], sidx)
    pltpu.sync_copy(dst_hbm.at[w], didx)
    plsc.subcore_barrier()

    def grp(g, carry):
        h = []
        for b in range(NBUF):
            j = g * NBUF + b
            h.append(pltpu.async_copy(hs_hbm.at[sidx.at[j]], rows.at[b], gsem))
        for hd in h:
            hd.wait()
        sc = []
        for b in range(NBUF):
            j = g * NBUF + b
            sc.append(pltpu.async_copy(rows.at[b], acc.at[didx.at[j]], ssem,
                                       add=True))
        for hd in sc:
            hd.wait()
        return carry

    lax.fori_loop(0, NGRP, grp, 0)
    plsc.subcore_barrier()

    @pl.when(s < 15)
    def _():
        pltpu.sync_copy(acc.at[pl.ds(s * 632, 632)], stage)
        pltpu.sync_copy(stage, out_hbm.at[c, pl.ds(s * 632, 632)])

    @pl.when(s == 15)
    def _():
        pltpu.sync_copy(acc.at[pl.ds(9480, 520)], stage.at[pl.ds(0, 520)])
        pltpu.sync_copy(stage.at[pl.ds(0, 520)], out_hbm.at[c, pl.ds(9480, 520)])


@functools.partial(
    pl.kernel,
    out_type=jax.ShapeDtypeStruct((EPAD, H), jnp.float32),
    mesh=_MESH,
    compiler_params=pltpu.CompilerParams(use_tc_tiling_on_sc=False),
    scratch_types=[
        pltpu.VMEM((NCH, CHK), jnp.int32),
        pltpu.VMEM((NBUF, CHK, H), jnp.float32),
        pltpu.SemaphoreType.DMA,
        pltpu.SemaphoreType.DMA,
    ],
)
def _gath_sc(h_hbm, src_hbm, out_hbm, sidx, rows, gsem, wsem):
    c = lax.axis_index("c")
    s = lax.axis_index("s")
    w = s * 2 + c
    pltpu.sync_copy(src_hbm.at[w], sidx)

    def grp(g, carry):
        hnd = []
        for b in range(NBUF):
            j = g * NBUF + b
            hnd.append(pltpu.async_copy(h_hbm.at[sidx.at[j]], rows.at[b], gsem))
        for hd in hnd:
            hd.wait()
        wh = []
        for b in range(NBUF):
            j = g * NBUF + b
            wh.append(pltpu.async_copy(
                rows.at[b], out_hbm.at[pl.ds(w * EPW + j * CHK, CHK)], wsem))
        for hd in wh:
            hd.wait()
        return carry

    lax.fori_loop(0, NGRP, grp, 0)


@functools.partial(
    pl.kernel,
    out_type=jax.ShapeDtypeStruct((2, N, H), jnp.float32),
    mesh=_MESH,
    compiler_params=pltpu.CompilerParams(use_tc_tiling_on_sc=False),
    scratch_types=[
        pltpu.VMEM((NCH, CHK), jnp.int32),
        pltpu.VMEM((NBUF, CHK, H), jnp.float32),
        pltpu.VMEM((632, H), jnp.float32),
        pltpu.VMEM_SHARED((NACC, H), jnp.float32),
        pltpu.SemaphoreType.DMA,
        pltpu.SemaphoreType.DMA,
    ],
)
def _scat_sc(rows_hbm, dst_hbm, zer_hbm, out_hbm,
             didx, rows, stage, acc, gsem, ssem):
    c = lax.axis_index("c")
    s = lax.axis_index("s")
    w = s * 2 + c
    pltpu.sync_copy(zer_hbm.at[pl.ds(s * 632, 632)], stage)
    pltpu.sync_copy(stage, acc.at[pl.ds(s * 632, 632)])
    pltpu.sync_copy(dst_hbm.at[w], didx)
    plsc.subcore_barrier()

    def grp(g, carry):
        hnd = []
        for b in range(NBUF):
            j = g * NBUF + b
            hnd.append(pltpu.async_copy(
                rows_hbm.at[pl.ds(w * EPW + j * CHK, CHK)], rows.at[b], gsem))
        for hd in hnd:
            hd.wait()
        sc = []
        for b in range(NBUF):
            j = g * NBUF + b
            sc.append(pltpu.async_copy(rows.at[b], acc.at[didx.at[j]], ssem,
                                       add=True))
        for hd in sc:
            hd.wait()
        return carry

    lax.fori_loop(0, NGRP, grp, 0)
    plsc.subcore_barrier()

    @pl.when(s < 15)
    def _():
        pltpu.sync_copy(acc.at[pl.ds(s * 632, 632)], stage)
        pltpu.sync_copy(stage, out_hbm.at[c, pl.ds(s * 632, 632)])

    @pl.when(s == 15)
    def _():
        pltpu.sync_copy(acc.at[pl.ds(9480, 520)], stage.at[pl.ds(0, 520)])
        pltpu.sync_copy(stage.at[pl.ds(0, 520)], out_hbm.at[c, pl.ds(9480, 520)])


@functools.partial(
    pl.kernel,
    out_type=(jax.ShapeDtypeStruct((2, NACC, 16), jnp.float32),
              jax.ShapeDtypeStruct((2, NACC, 16), jnp.float32)),
    mesh=_MESH,
    compiler_params=pltpu.CompilerParams(use_tc_tiling_on_sc=False),
    scratch_types=[
        pltpu.VMEM((NCH, CHK), jnp.int32),
        pltpu.VMEM((NCH, CHK), jnp.int32),
        pltpu.VMEM((CHK, 16), jnp.float32),
        pltpu.VMEM((632, 16), jnp.float32),
        pltpu.VMEM_SHARED((NACC, 16), jnp.float32),
        pltpu.VMEM_SHARED((NACC, 16), jnp.float32),
        pltpu.SemaphoreType.DMA,
    ],
)
def _deg_sc(dst_hbm, cid_hbm, ones_hbm, zer_hbm, odeg_hbm, ocnt_hbm,
            didx, cidx, ones, stage, adeg, acnt, sem):
    c = lax.axis_index("c")
    s = lax.axis_index("s")
    w = s * 2 + c
    pltpu.sync_copy(zer_hbm.at[pl.ds(s * 632, 632)], stage)
    pltpu.sync_copy(stage, adeg.at[pl.ds(s * 632, 632)])
    pltpu.sync_copy(stage, acnt.at[pl.ds(s * 632, 632)])
    pltpu.sync_copy(ones_hbm, ones)
    pltpu.sync_copy(dst_hbm.at[w], didx)
    pltpu.sync_copy(cid_hbm.at[w], cidx)
    plsc.subcore_barrier()

    def grp(g, carry):
        hnd = []
        for b in range(NBUF):
            j = g * NBUF + b
            hnd.append(pltpu.async_copy(ones, adeg.at[didx.at[j]], sem,
                                        add=True))
            hnd.append(pltpu.async_copy(ones, acnt.at[cidx.at[j]], sem,
                                        add=True))
        for hd in hnd:
            hd.wait()
        return carry

    lax.fori_loop(0, NGRP, grp, 0)
    plsc.subcore_barrier()
    pltpu.sync_copy(adeg.at[pl.ds(s * 632, 632)], stage)
    pltpu.sync_copy(stage, odeg_hbm.at[c, pl.ds(s * 632, 632)])
    pltpu.sync_copy(acnt.at[pl.ds(s * 632, 632)], stage)
    pltpu.sync_copy(stage, ocnt_hbm.at[c, pl.ds(s * 632, 632)])


BLK = 2048


def _msg_body(ea_ref, xs_ref, A1_ref, a1_ref, A2_ref, a2r_ref, out_ref):
    h8 = jnp.maximum(
        jnp.dot(ea_ref[...], A1_ref[...],
                preferred_element_type=jnp.float32) + a1_ref[...], 0.0)
    xs = xs_ref[...]
    acc = jnp.dot(xs, a2r_ref[...], preferred_element_type=jnp.float32)
    for j in range(8):
        acc += h8[:, j:j + 1] * jnp.dot(xs, A2_ref[j],
                                        preferred_element_type=jnp.float32)
    out_ref[...] = acc


def _msg_tc(ea_pad, xs, A1, a1, A2r3, a2r):
    return pl.pallas_call(
        _msg_body,
        grid=(EPAD // BLK,),
        in_specs=[
            pl.BlockSpec((BLK, 4), lambda i: (i, 0)),
            pl.BlockSpec((BLK, H), lambda i: (i, 0)),
            pl.BlockSpec((4, 8), lambda i: (0, 0)),
            pl.BlockSpec((1, 8), lambda i: (0, 0)),
            pl.BlockSpec((8, H, H), lambda i: (0, 0, 0)),
            pl.BlockSpec((H, H), lambda i: (0, 0)),
        ],
        out_specs=pl.BlockSpec((BLK, H), lambda i: (i, 0)),
        out_shape=jax.ShapeDtypeStruct((EPAD, H), jnp.float32),
    )(ea_pad, xs, A1, a1, A2r3, a2r)


NPD = 10240
NPW = 320
DCH = 64


@functools.partial(
    pl.kernel,
    out_type=jax.ShapeDtypeStruct((NPD, H), jnp.float32),
    mesh=_MESH,
    compiler_params=pltpu.CompilerParams(use_tc_tiling_on_sc=False),
    scratch_types=[
        pltpu.VMEM((S, NPW), jnp.int32),
        pltpu.VMEM((2, DCH, H), jnp.float32),
        pltpu.SemaphoreType.DMA,
        pltpu.SemaphoreType.DMA,
    ],
)
def _dec_sc(lo_hbm, nbT_hbm, g_hbm, nbbuf, rows, sem, osem):
    c = lax.axis_index("c")
    s = lax.axis_index("s")
    w = s * 2 + c
    base = w * NPW
    pltpu.sync_copy(nbT_hbm.at[:, pl.ds(base, NPW)], nbbuf)
    ohnd = []
    for t in range(NPW // DCH):
        b = t % 2
        pltpu.async_copy(lo_hbm.at[nbbuf.at[0, pl.ds(t * DCH, DCH)]],
                         rows.at[b], sem).wait()
        hnd = [pltpu.async_copy(lo_hbm.at[nbbuf.at[k, pl.ds(t * DCH, DCH)]],
                                rows.at[b], sem, add=True)
               for k in range(1, S)]
        for hd in hnd:
            hd.wait()
        ohnd.append(pltpu.async_copy(
            rows.at[b], g_hbm.at[pl.ds(base + t * DCH, DCH)], osem))
        if len(ohnd) == 2:
            ohnd.pop(0).wait()
    for hd in ohnd:
        hd.wait()


def _gru_body(gi_ref, WhT_ref, bh_ref, Wl_ref, bl_ref, lo_ref):
    WhT = WhT_ref[...]
    bh2 = bh_ref[...]
    Wl = Wl_ref[...]
    bl2 = bl_ref[...]

    def step(h, g):
        gh = jnp.dot(h, WhT, preferred_element_type=jnp.float32) + bh2
        r = jax.nn.sigmoid(g[:, :H] + gh[:, :H])
        zg = jax.nn.sigmoid(g[:, H:2 * H] + gh[:, H:2 * H])
        ng = jnp.tanh(g[:, 2 * H:] + r * gh[:, 2 * H:])
        return (1.0 - zg) * ng + zg * h

    def warm(s_, h):
        return step(h, gi_ref[s_])

    h = lax.fori_loop(0, K, warm, jnp.zeros((C, H), jnp.float32))
    h = jnp.concatenate([jnp.zeros((1, H), jnp.float32), h[:C - 1, :]], axis=0)

    def main(s_, h):
        h = step(h, gi_ref[s_])
        lo_ref[s_] = jnp.dot(h, Wl, preferred_element_type=jnp.float32) + bl2
        return h

    lax.fori_loop(0, K, main, h)


def _gru_tc(giT, WhT, bh2, Wl, bl2):
    return pl.pallas_call(
        _gru_body,
        out_shape=jax.ShapeDtypeStruct((K, C, H), jnp.float32),
    )(giT, WhT, bh2, Wl, bl2)


def kernel(x, eis, eas, idxs, ptrs, W1, b1, W2, b2, W3, b3, A1, a1, A2, a2,
           root, bc4, Wi, bi, Wh, bh, Wl, bl, Wd, bd):
    ei = eis[0]
    ea = eas[0]
    idx = idxs[0]
    ptr = ptrs[0]
    src = ei[0]
    dst = ei[1]
    n = x.shape[0]

    pad = EPAD - E
    src_p = jnp.concatenate([src, jnp.zeros((pad,), src.dtype)])
    dst_p = jnp.concatenate([dst, jnp.full((pad,), DUMP, dst.dtype)])
    src3 = src_p.reshape(NW, NCH, CHK).astype(jnp.int32)
    dst3 = dst_p.reshape(NW, NCH, CHK).astype(jnp.int32)
    zer = jnp.zeros((NACC, H), jnp.float32)

    cid = jnp.where(src_p == dst_p, src_p, DUMP)
    cid3 = cid.reshape(NW, NCH, CHK).astype(jnp.int32)
    odeg, ocnt = _deg_sc(dst3, cid3, jnp.ones((CHK, 16), jnp.float32),
                         jnp.zeros((NACC, 16), jnp.float32))
    indeg = (odeg[0, :n, 0] + odeg[1, :n, 0])
    cnt = (ocnt[0, :n, 0] + ocnt[1, :n, 0])
    loopw = (cnt == 0).astype(jnp.float32)
    deg = indeg + loopw
    dinv = jax.lax.rsqrt(deg)
    diag = loopw * dinv * dinv

    def prop(h):
        hs = dinv[:, None] * h
        p = _prop_sc(hs, src3, dst3, zer)
        return dinv[:, None] * (p[0] + p[1]) + diag[:, None] * h

    h1 = x @ W1
    z1 = prop(h1) + b1
    h2 = z1 @ W2
    z2 = jax.nn.relu(prop(h2) + b2)
    h3 = z2 @ W3
    z3 = jax.nn.relu(prop(h3) + b3)

    ea_pad = jnp.concatenate([ea, jnp.zeros((pad, ea.shape[1]), ea.dtype)])
    xs_pad = _gath_sc(z3, src3)
    msg_pad = _msg_tc(ea_pad, xs_pad, A1, a1.reshape(1, 8),
                      A2.reshape(8, H, H), a2.reshape(H, H))
    s4p = _scat_sc(msg_pad, dst3, zer)
    s4 = s4p[0] + s4p[1]
    aggr = s4 / jnp.maximum(indeg, 1.0)[:, None]
    z4 = jnp.tanh(z3 @ root + aggr + bc4)

    gi = z4 @ Wi.T + bi
    gi = jnp.concatenate(
        [gi, jnp.zeros((C * K - n, 3 * H), jnp.float32)], axis=0)
    giT = gi.reshape(C, K, 3 * H).transpose(1, 0, 2)
    loT = _gru_tc(giT, Wh.T, bh.reshape(1, 3 * H), Wl, bl.reshape(1, H))
    lo_pad = loT.transpose(1, 0, 2).reshape(C * K, H)
    lo = lo_pad[:n]

    deg_s = idx[1:] - idx[:-1]
    r = jax.random.randint(jax.random.fold_in(jax.random.key(42), 0),
                           (n, S), 0, 1 << 30)
    off = r % jnp.maximum(deg_s, 1)[:, None]
    nb = ptr[idx[:-1][:, None] + off]
    nbT = jnp.concatenate(
        [nb.astype(jnp.int32).T, jnp.zeros((S, NPD - n), jnp.int32)], axis=1)
    agg = _dec_sc(lo_pad, nbT)[:n]
    zagg = jnp.where((deg_s > 0)[:, None], (agg + lo) / (S + 1.0), lo)
    out0 = jax.nn.softmax(zagg @ Wd + bd, axis=1)
    return jnp.stack([out0])
